# baseline (speedup 1.0000x reference)
import jax
import jax.numpy as jnp
from jax.experimental import pallas as pl
from jax.experimental.pallas import tpu as pltpu

GROUP_SIZE = 16
MIN_SIGMA = 1e-4
MBLK = 128


def _body(p_ref, pt4_ref, q_ref, inv_sigma_ref, out_ref):
    p = p_ref[0]
    q = q_ref[0]
    px = p[0:1, :]
    py = p[1:2, :]
    pz = p[2:3, :]
    qx = q[:, 0:1]
    qy = q[:, 1:2]
    qz = q[:, 2:3]
    p2 = px * px + py * py + pz * pz
    q2 = qx * qx + qy * qy + qz * qz
    bf = jnp.bfloat16
    f32 = jnp.float32
    csel = jax.lax.dot_general(
        q.astype(bf), p.astype(bf),
        (((1,), (0,)), ((), ())),
        preferred_element_type=f32)
    dsel = q2 - 2.0 * csel + p2
    c = qx * px + qy * py + qz * pz
    d2 = q2 - 2.0 * c + p2

    inf = jnp.float32(jnp.inf)
    t = jnp.min(dsel, axis=1, keepdims=True)
    for _ in range(GROUP_SIZE - 1):
        t = jnp.min(jnp.where(dsel > t, dsel, inf), axis=1, keepdims=True)

    mask = dsel <= t
    dmin = jnp.min(jnp.where(mask, d2, inf), axis=1, keepdims=True)
    inv_sigma = inv_sigma_ref[0]
    w = jnp.where(mask, jnp.exp((dmin - d2) * inv_sigma), 0.0)
    nd = jax.lax.dot_general(
        w, pt4_ref[0],
        (((1,), (0,)), ((), ())),
        precision=jax.lax.Precision.HIGHEST,
        preferred_element_type=f32)
    r = 1.0 / nd[:, 3:4]
    out_ref[0] = nd[:, 0:3] * r


def _build(B, N, M, interpret=False):
    grid = (B, M // MBLK)
    return pl.pallas_call(
        _body,
        grid=grid,
        in_specs=[
            pl.BlockSpec((1, 3, N), lambda b, j: (b, 0, 0)),
            pl.BlockSpec((1, N, 4), lambda b, j: (b, 0, 0)),
            pl.BlockSpec((1, MBLK, 3), lambda b, j: (b, j, 0)),
            pl.BlockSpec(memory_space=pltpu.SMEM),
        ],
        out_specs=pl.BlockSpec((1, MBLK, 3), lambda b, j: (b, j, 0)),
        out_shape=jax.ShapeDtypeStruct((B, M, 3), jnp.float32),
        interpret=interpret,
    )


def kernel(point_cloud, query_cloud, temperature):
    B, _, N = point_cloud.shape
    M = query_cloud.shape[2]
    pt = jnp.transpose(point_cloud, (0, 2, 1))
    pt4 = jnp.concatenate(
        [pt, jnp.ones((B, N, 1), jnp.float32)], axis=2)
    qt = jnp.transpose(query_cloud, (0, 2, 1))
    sigma = jnp.maximum(temperature ** 2, jnp.asarray(MIN_SIGMA, jnp.float32))
    inv_sigma = (1.0 / sigma).reshape(1).astype(jnp.float32)
    out = _build(B, N, M)(point_cloud, pt4, qt, inv_sigma)
    return jnp.transpose(out, (0, 2, 1))


# speedup vs baseline: 1.0403x; 1.0403x over previous
import jax
import jax.numpy as jnp
from jax import lax
from jax.experimental import pallas as pl
from jax.experimental.pallas import tpu as pltpu
from jax.experimental.pallas import tpu_sc as plsc

GROUP_SIZE = 16
MIN_SIGMA = 1e-4
L = 16
G = 4
QW = 256
CAP = 1024
BSTRIDE = CAP + 2 * L
F32 = jnp.float32
I32 = jnp.int32
INF = float("inf")


def _sc_build(B, N, M):
    NCHUNK = N // L
    mesh = plsc.VectorSubcoreMesh(core_axis_name="c", subcore_axis_name="s")

    def body(pc_hbm, qc_hbm, isig_hbm, out_hbm,
             pbx, pby, pbz, p2v, stage, qex, qey, qez, q2vm,
             qrx_v, qry_v, qrz_v, isig_v,
             bufd, bufi, topd, topi, gidx, gax, gay, gaz,
             pwx, pwy, pwz, dwork, outx, outy, outz, sem):
        wid = lax.axis_index("s") * 2 + lax.axis_index("c")
        b = wid // 4
        qoff = (wid % 4) * QW
        pbase = b * 3 * N
        qbase = b * 3 * M + qoff

        def rne16(x):
            bi = plsc.bitcast(x, I32)
            r = bi + jnp.int32(0x7FFF) + jnp.bitwise_and(
                lax.shift_right_logical(bi, 16), jnp.int32(1))
            r = jnp.bitwise_and(r, jnp.int32(-65536))
            return plsc.bitcast(r, F32)

        pltpu.sync_copy(pc_hbm.at[pl.ds(pbase, N)], pbx)
        pltpu.sync_copy(pc_hbm.at[pl.ds(pbase + N, N)], pby)
        pltpu.sync_copy(pc_hbm.at[pl.ds(pbase + 2 * N, N)], pbz)

        def p2_round(i, _):
            vx = pbx[pl.ds(i * L, L)]
            vy = pby[pl.ds(i * L, L)]
            vz = pbz[pl.ds(i * L, L)]
            p2v[pl.ds(i * L, L)] = vx * vx + vy * vy + vz * vz
            pbx[pl.ds(i * L, L)] = rne16(vx)
            pby[pl.ds(i * L, L)] = rne16(vy)
            pbz[pl.ds(i * L, L)] = rne16(vz)
            return 0
        lax.fori_loop(0, NCHUNK, p2_round, 0)

        pltpu.sync_copy(qc_hbm.at[pl.ds(qbase, QW)], qex)
        pltpu.sync_copy(qc_hbm.at[pl.ds(qbase + M, QW)], qey)
        pltpu.sync_copy(qc_hbm.at[pl.ds(qbase + 2 * M, QW)], qez)
        pltpu.sync_copy(qc_hbm.at[pl.ds(qbase, QW)], qrx_v)
        pltpu.sync_copy(qc_hbm.at[pl.ds(qbase + M, QW)], qry_v)
        pltpu.sync_copy(qc_hbm.at[pl.ds(qbase + 2 * M, QW)], qrz_v)
        pltpu.sync_copy(isig_hbm, isig_v)

        def q_round(i, _):
            qrx_v[pl.ds(i * L, L)] = rne16(qrx_v[pl.ds(i * L, L)])
            qry_v[pl.ds(i * L, L)] = rne16(qry_v[pl.ds(i * L, L)])
            qrz_v[pl.ds(i * L, L)] = rne16(qrz_v[pl.ds(i * L, L)])
            return 0
        lax.fori_loop(0, QW // L, q_round, 0)

        def q2_body(i, _):
            vx = qex[pl.ds(i * L, L)]
            vy = qey[pl.ds(i * L, L)]
            vz = qez[pl.ds(i * L, L)]
            q2vm[pl.ds(i * L, L)] = vx * vx + vy * vy + vz * vz
            return 0
        lax.fori_loop(0, QW // L, q2_body, 0)

        def group_body(g, _):
            base = g * G
            zi = jnp.zeros((L,), I32)
            qx = [plsc.load_gather(qrx_v, [zi + (base + j)]) for j in range(G)]
            qy = [plsc.load_gather(qry_v, [zi + (base + j)]) for j in range(G)]
            qz = [plsc.load_gather(qrz_v, [zi + (base + j)]) for j in range(G)]
            q2 = [plsc.load_gather(q2vm, [zi + (base + j)]) for j in range(G)]
            for j in range(G):
                topd[pl.ds(j * L, L)] = jnp.full((L,), INF, F32)
                topi[pl.ds(j * L, L)] = jnp.zeros((L,), I32)

            def chunk_body(ci, carry):
                offs = list(carry[:G])
                ts = list(carry[G:])
                vx = pbx[pl.ds(ci * L, L)]
                vy = pby[pl.ds(ci * L, L)]
                vz = pbz[pl.ds(ci * L, L)]
                vp2 = p2v[pl.ds(ci * L, L)]
                iv = lax.iota(I32, L) + ci * L
                for j in range(G):
                    cj = qx[j] * vx + qy[j] * vy + qz[j] * vz
                    dj = (q2[j] - 2.0 * cj) + vp2
                    mj = dj < ts[j]
                    cnt = jnp.max(plsc.all_reduce_population_count(mj))
                    plsc.store_compressed(
                        bufd.at[pl.ds(j * BSTRIDE + offs[j], L)], dj, mask=mj)
                    plsc.store_compressed(
                        bufi.at[pl.ds(j * BSTRIDE + offs[j], L)], iv, mask=mj)
                    offs[j] = offs[j] + cnt
                return tuple(offs) + tuple(ts)

            def drain_all(carry):
                offs = carry[:G]
                ts = []
                for j in range(G):
                    Tj = topd[pl.ds(j * L, L)]
                    Ij = topi[pl.ds(j * L, L)]

                    def dcond(st):
                        return st[0] < offs[j]

                    def dbody(st):
                        k, T, I = st
                        v = bufd[pl.ds(j * BSTRIDE + k, L)]
                        ivv = bufi[pl.ds(j * BSTRIDE + k, L)]
                        lane = lax.iota(I32, L)
                        v = jnp.where(lane < (offs[j] - k), v, INF)
                        vs, ivs = plsc.sort_key_val(v, ivv)
                        vr = lax.rev(vs, (0,))
                        ir = lax.rev(ivs, (0,))
                        sel = T <= vr
                        Lv = jnp.where(sel, T, vr)
                        Li = jnp.where(sel, I, ir)
                        T2, I2 = plsc.sort_key_val(Lv, Li)
                        return (k + L, T2, I2)

                    _, Tj, Ij = lax.while_loop(dcond, dbody, (0, Tj, Ij))
                    topd[pl.ds(j * L, L)] = Tj
                    topi[pl.ds(j * L, L)] = Ij
                    ts.append(jnp.max(Tj))
                return (jnp.int32(0),) * G + tuple(ts)

            carry = (jnp.int32(0),) * G + (jnp.float32(INF),) * G
            for (s0, s1) in ((0, 1), (1, 8), (8, 64)):
                carry = lax.fori_loop(s0, s1, chunk_body, carry)
                carry = drain_all(carry)

            def seg_body(s, carry):
                carry = lax.fori_loop(64 + s * 64, 128 + s * 64,
                                      chunk_body, carry)
                return drain_all(carry)
            carry = lax.fori_loop(0, (NCHUNK - 64) // 64, seg_body, carry)

            for j in range(G):
                gidx[pl.ds((base + j) * L, L)] = topi[pl.ds(j * L, L)]
            return 0
        lax.fori_loop(0, QW // G, group_body, 0)

        for c, dstbuf in enumerate((gax, gay, gaz)):
            pltpu.sync_copy(pc_hbm.at[pl.ds(pbase + c * N, N)], stage)

            def g_body(i, _):
                idxv = gidx[pl.ds(i * L, L)]
                dstbuf[pl.ds(i * L, L)] = plsc.load_gather(stage, [idxv])
                return 0
            lax.fori_loop(0, QW * L // L, g_body, 0)

        def fin_body(fg, _):
            qxv = qex[pl.ds(fg * L, L)]
            qyv = qey[pl.ds(fg * L, L)]
            qzv = qez[pl.ds(fg * L, L)]
            lane = lax.iota(I32, L)
            rowb = fg * QW + lane * L
            isg = isig_v[pl.ds(0, L)]
            minv = jnp.full((L,), INF, F32)
            for k in range(GROUP_SIZE):
                rows = rowb + k
                gx = plsc.load_gather(gax, [rows])
                gy = plsc.load_gather(gay, [rows])
                gz = plsc.load_gather(gaz, [rows])
                dx = gx - qxv
                dy = gy - qyv
                dz = gz - qzv
                dk = dx * dx + dy * dy + dz * dz
                pwx[pl.ds(k * L, L)] = gx
                pwy[pl.ds(k * L, L)] = gy
                pwz[pl.ds(k * L, L)] = gz
                dwork[pl.ds(k * L, L)] = dk
                minv = jnp.minimum(minv, dk)
            den = jnp.zeros((L,), F32)
            ox = jnp.zeros((L,), F32)
            oy = jnp.zeros((L,), F32)
            oz = jnp.zeros((L,), F32)
            for k in range(GROUP_SIZE):
                ek = jnp.exp((minv - dwork[pl.ds(k * L, L)]) * isg)
                den = den + ek
                ox = ox + ek * pwx[pl.ds(k * L, L)]
                oy = oy + ek * pwy[pl.ds(k * L, L)]
                oz = oz + ek * pwz[pl.ds(k * L, L)]
            outx[pl.ds(fg * L, L)] = ox / den
            outy[pl.ds(fg * L, L)] = oy / den
            outz[pl.ds(fg * L, L)] = oz / den
            return 0
        lax.fori_loop(0, QW // L, fin_body, 0)

        pltpu.sync_copy(outx, out_hbm.at[pl.ds(qbase, QW)])
        pltpu.sync_copy(outy, out_hbm.at[pl.ds(qbase + M, QW)])
        pltpu.sync_copy(outz, out_hbm.at[pl.ds(qbase + 2 * M, QW)])

    return pl.kernel(
        body,
        out_type=jax.ShapeDtypeStruct((B * 3 * M,), F32),
        mesh=mesh,
        compiler_params=pltpu.CompilerParams(
            needs_layout_passes=False, use_tc_tiling_on_sc=False),
        scratch_types=[
            pltpu.VMEM((N,), F32),
            pltpu.VMEM((N,), F32),
            pltpu.VMEM((N,), F32),
            pltpu.VMEM((N,), F32),
            pltpu.VMEM((N,), F32),
            pltpu.VMEM((QW,), F32),
            pltpu.VMEM((QW,), F32),
            pltpu.VMEM((QW,), F32),
            pltpu.VMEM((QW,), F32),
            pltpu.VMEM((QW,), F32),
            pltpu.VMEM((QW,), F32),
            pltpu.VMEM((QW,), F32),
            pltpu.VMEM((L,), F32),
            pltpu.VMEM((G * BSTRIDE,), F32),
            pltpu.VMEM((G * BSTRIDE,), I32),
            pltpu.VMEM((G * L,), F32),
            pltpu.VMEM((G * L,), I32),
            pltpu.VMEM((QW * L,), I32),
            pltpu.VMEM((QW * L,), F32),
            pltpu.VMEM((QW * L,), F32),
            pltpu.VMEM((QW * L,), F32),
            pltpu.VMEM((GROUP_SIZE * L,), F32),
            pltpu.VMEM((GROUP_SIZE * L,), F32),
            pltpu.VMEM((GROUP_SIZE * L,), F32),
            pltpu.VMEM((GROUP_SIZE * L,), F32),
            pltpu.VMEM((QW,), F32),
            pltpu.VMEM((QW,), F32),
            pltpu.VMEM((QW,), F32),
            pltpu.SemaphoreType.DMA,
        ],
    )


def kernel(point_cloud, query_cloud, temperature):
    B, _, N = point_cloud.shape
    M = query_cloud.shape[2]
    sigma = jnp.maximum(temperature ** 2, jnp.asarray(MIN_SIGMA, F32))
    isig = jnp.full((L,), 1.0, F32) / sigma
    out = _sc_build(B, N, M)(
        point_cloud.reshape(-1), query_cloud.reshape(-1), isig)
    return out.reshape(B, 3, M)


# speedup vs baseline: 2.0260x; 1.9474x over previous
import jax
import jax.numpy as jnp
from jax import lax
from jax.experimental import pallas as pl
from jax.experimental.pallas import tpu as pltpu
from jax.experimental.pallas import tpu_sc as plsc

GROUP_SIZE = 16
MIN_SIGMA = 1e-4
L = 16
G = 4
CAP = 1024
BSTRIDE = CAP + 2 * L
F32 = jnp.float32
I32 = jnp.int32
INF = float("inf")


def _sc_build(B, N, M, QSC):
    NCHUNK = N // L
    QW = QSC // 4
    mesh = plsc.VectorSubcoreMesh(core_axis_name="c", subcore_axis_name="s")

    def body(pc_hbm, qc_hbm, isig_hbm, out_hbm,
             pbx, pby, pbz, p2v, stage, qex, qey, qez, q2vm,
             qrx_v, qry_v, qrz_v, isig_v,
             bufd, bufi, topd, topi, gidx, gax, gay, gaz,
             pwx, pwy, pwz, dwork, outx, outy, outz, sem):
        wid = lax.axis_index("s") * 2 + lax.axis_index("c")
        b = wid // 4
        qoff = (wid % 4) * QW
        pbase = b * 3 * N
        qbase = b * 3 * M + qoff
        obase = b * 3 * QSC + qoff

        def rne16(x):
            bi = plsc.bitcast(x, I32)
            r = bi + jnp.int32(0x7FFF) + jnp.bitwise_and(
                lax.shift_right_logical(bi, 16), jnp.int32(1))
            r = jnp.bitwise_and(r, jnp.int32(-65536))
            return plsc.bitcast(r, F32)

        pltpu.sync_copy(pc_hbm.at[pl.ds(pbase, N)], pbx)
        pltpu.sync_copy(pc_hbm.at[pl.ds(pbase + N, N)], pby)
        pltpu.sync_copy(pc_hbm.at[pl.ds(pbase + 2 * N, N)], pbz)

        def p2_round(i, _):
            vx = pbx[pl.ds(i * L, L)]
            vy = pby[pl.ds(i * L, L)]
            vz = pbz[pl.ds(i * L, L)]
            p2v[pl.ds(i * L, L)] = vx * vx + vy * vy + vz * vz
            pbx[pl.ds(i * L, L)] = rne16(vx)
            pby[pl.ds(i * L, L)] = rne16(vy)
            pbz[pl.ds(i * L, L)] = rne16(vz)
            return 0
        lax.fori_loop(0, NCHUNK, p2_round, 0)

        pltpu.sync_copy(qc_hbm.at[pl.ds(qbase, QW)], qex)
        pltpu.sync_copy(qc_hbm.at[pl.ds(qbase + M, QW)], qey)
        pltpu.sync_copy(qc_hbm.at[pl.ds(qbase + 2 * M, QW)], qez)
        pltpu.sync_copy(qc_hbm.at[pl.ds(qbase, QW)], qrx_v)
        pltpu.sync_copy(qc_hbm.at[pl.ds(qbase + M, QW)], qry_v)
        pltpu.sync_copy(qc_hbm.at[pl.ds(qbase + 2 * M, QW)], qrz_v)
        pltpu.sync_copy(isig_hbm, isig_v)

        def q_round(i, _):
            qrx_v[pl.ds(i * L, L)] = rne16(qrx_v[pl.ds(i * L, L)])
            qry_v[pl.ds(i * L, L)] = rne16(qry_v[pl.ds(i * L, L)])
            qrz_v[pl.ds(i * L, L)] = rne16(qrz_v[pl.ds(i * L, L)])
            return 0
        lax.fori_loop(0, QW // L, q_round, 0)

        def q2_body(i, _):
            vx = qex[pl.ds(i * L, L)]
            vy = qey[pl.ds(i * L, L)]
            vz = qez[pl.ds(i * L, L)]
            q2vm[pl.ds(i * L, L)] = vx * vx + vy * vy + vz * vz
            return 0
        lax.fori_loop(0, QW // L, q2_body, 0)

        def group_body(g, _):
            base = g * G
            zi = jnp.zeros((L,), I32)
            qx = [plsc.load_gather(qrx_v, [zi + (base + j)]) for j in range(G)]
            qy = [plsc.load_gather(qry_v, [zi + (base + j)]) for j in range(G)]
            qz = [plsc.load_gather(qrz_v, [zi + (base + j)]) for j in range(G)]
            q2 = [plsc.load_gather(q2vm, [zi + (base + j)]) for j in range(G)]
            for j in range(G):
                topd[pl.ds(j * L, L)] = jnp.full((L,), INF, F32)
                topi[pl.ds(j * L, L)] = jnp.zeros((L,), I32)

            def chunk_body(ci, carry):
                offs = list(carry[:G])
                ts = list(carry[G:])
                vx = pbx[pl.ds(ci * L, L)]
                vy = pby[pl.ds(ci * L, L)]
                vz = pbz[pl.ds(ci * L, L)]
                vp2 = p2v[pl.ds(ci * L, L)]
                iv = lax.iota(I32, L) + ci * L
                for j in range(G):
                    cj = qx[j] * vx + qy[j] * vy + qz[j] * vz
                    dj = (q2[j] - 2.0 * cj) + vp2
                    mj = dj < ts[j]
                    cnt = jnp.max(plsc.all_reduce_population_count(mj))
                    plsc.store_compressed(
                        bufd.at[pl.ds(j * BSTRIDE + offs[j], L)], dj, mask=mj)
                    plsc.store_compressed(
                        bufi.at[pl.ds(j * BSTRIDE + offs[j], L)], iv, mask=mj)
                    offs[j] = offs[j] + cnt
                return tuple(offs) + tuple(ts)

            def drain_all(carry):
                offs = carry[:G]
                ts = []
                for j in range(G):
                    Tj = topd[pl.ds(j * L, L)]
                    Ij = topi[pl.ds(j * L, L)]

                    def dcond(st):
                        return st[0] < offs[j]

                    def dbody(st):
                        k, T, I = st
                        v = bufd[pl.ds(j * BSTRIDE + k, L)]
                        ivv = bufi[pl.ds(j * BSTRIDE + k, L)]
                        lane = lax.iota(I32, L)
                        v = jnp.where(lane < (offs[j] - k), v, INF)
                        vs, ivs = plsc.sort_key_val(v, ivv)
                        vr = lax.rev(vs, (0,))
                        ir = lax.rev(ivs, (0,))
                        sel = T <= vr
                        Lv = jnp.where(sel, T, vr)
                        Li = jnp.where(sel, I, ir)
                        T2, I2 = plsc.sort_key_val(Lv, Li)
                        return (k + L, T2, I2)

                    _, Tj, Ij = lax.while_loop(dcond, dbody, (0, Tj, Ij))
                    topd[pl.ds(j * L, L)] = Tj
                    topi[pl.ds(j * L, L)] = Ij
                    ts.append(jnp.max(Tj))
                return (jnp.int32(0),) * G + tuple(ts)

            carry = (jnp.int32(0),) * G + (jnp.float32(INF),) * G
            for (s0, s1) in ((0, 1), (1, 8), (8, 64)):
                carry = lax.fori_loop(s0, s1, chunk_body, carry)
                carry = drain_all(carry)

            def seg_body(s, carry):
                carry = lax.fori_loop(64 + s * 64, 128 + s * 64,
                                      chunk_body, carry)
                return drain_all(carry)
            carry = lax.fori_loop(0, (NCHUNK - 64) // 64, seg_body, carry)

            for j in range(G):
                gidx[pl.ds((base + j) * L, L)] = topi[pl.ds(j * L, L)]
            return 0
        lax.fori_loop(0, QW // G, group_body, 0)

        for c, dstbuf in enumerate((gax, gay, gaz)):
            pltpu.sync_copy(pc_hbm.at[pl.ds(pbase + c * N, N)], stage)

            def g_body(i, _):
                idxv = gidx[pl.ds(i * L, L)]
                dstbuf[pl.ds(i * L, L)] = plsc.load_gather(stage, [idxv])
                return 0
            lax.fori_loop(0, QW * L // L, g_body, 0)

        def fin_body(fg, _):
            qxv = qex[pl.ds(fg * L, L)]
            qyv = qey[pl.ds(fg * L, L)]
            qzv = qez[pl.ds(fg * L, L)]
            lane = lax.iota(I32, L)
            rowb = fg * (L * L) + lane * L
            isg = isig_v[pl.ds(0, L)]
            minv = jnp.full((L,), INF, F32)
            for k in range(GROUP_SIZE):
                rows = rowb + k
                gx = plsc.load_gather(gax, [rows])
                gy = plsc.load_gather(gay, [rows])
                gz = plsc.load_gather(gaz, [rows])
                dx = gx - qxv
                dy = gy - qyv
                dz = gz - qzv
                dk = dx * dx + dy * dy + dz * dz
                pwx[pl.ds(k * L, L)] = gx
                pwy[pl.ds(k * L, L)] = gy
                pwz[pl.ds(k * L, L)] = gz
                dwork[pl.ds(k * L, L)] = dk
                minv = jnp.minimum(minv, dk)
            den = jnp.zeros((L,), F32)
            ox = jnp.zeros((L,), F32)
            oy = jnp.zeros((L,), F32)
            oz = jnp.zeros((L,), F32)
            for k in range(GROUP_SIZE):
                ek = jnp.exp((minv - dwork[pl.ds(k * L, L)]) * isg)
                den = den + ek
                ox = ox + ek * pwx[pl.ds(k * L, L)]
                oy = oy + ek * pwy[pl.ds(k * L, L)]
                oz = oz + ek * pwz[pl.ds(k * L, L)]
            outx[pl.ds(fg * L, L)] = ox / den
            outy[pl.ds(fg * L, L)] = oy / den
            outz[pl.ds(fg * L, L)] = oz / den
            return 0
        lax.fori_loop(0, QW // L, fin_body, 0)

        pltpu.sync_copy(outx, out_hbm.at[pl.ds(obase, QW)])
        pltpu.sync_copy(outy, out_hbm.at[pl.ds(obase + QSC, QW)])
        pltpu.sync_copy(outz, out_hbm.at[pl.ds(obase + 2 * QSC, QW)])

    return pl.kernel(
        body,
        out_type=jax.ShapeDtypeStruct((B * 3 * QSC,), F32),
        mesh=mesh,
        compiler_params=pltpu.CompilerParams(
            needs_layout_passes=False, use_tc_tiling_on_sc=False),
        scratch_types=[
            pltpu.VMEM((N,), F32),
            pltpu.VMEM((N,), F32),
            pltpu.VMEM((N,), F32),
            pltpu.VMEM((N,), F32),
            pltpu.VMEM((N,), F32),
            pltpu.VMEM((QW,), F32),
            pltpu.VMEM((QW,), F32),
            pltpu.VMEM((QW,), F32),
            pltpu.VMEM((QW,), F32),
            pltpu.VMEM((QW,), F32),
            pltpu.VMEM((QW,), F32),
            pltpu.VMEM((QW,), F32),
            pltpu.VMEM((L,), F32),
            pltpu.VMEM((G * BSTRIDE,), F32),
            pltpu.VMEM((G * BSTRIDE,), I32),
            pltpu.VMEM((G * L,), F32),
            pltpu.VMEM((G * L,), I32),
            pltpu.VMEM((QW * L,), I32),
            pltpu.VMEM((QW * L,), F32),
            pltpu.VMEM((QW * L,), F32),
            pltpu.VMEM((QW * L,), F32),
            pltpu.VMEM((GROUP_SIZE * L,), F32),
            pltpu.VMEM((GROUP_SIZE * L,), F32),
            pltpu.VMEM((GROUP_SIZE * L,), F32),
            pltpu.VMEM((GROUP_SIZE * L,), F32),
            pltpu.VMEM((QW,), F32),
            pltpu.VMEM((QW,), F32),
            pltpu.VMEM((QW,), F32),
            pltpu.SemaphoreType.DMA,
        ],
    )


MBLK = 128


def _tc_body(p_ref, q_ref, inv_sigma_ref, out_ref):
    p = p_ref[0]
    q = q_ref[0]
    px = p[0:1, :]
    py = p[1:2, :]
    pz = p[2:3, :]
    qx = q[:, 0:1]
    qy = q[:, 1:2]
    qz = q[:, 2:3]
    p2 = px * px + py * py + pz * pz
    q2 = qx * qx + qy * qy + qz * qz
    bf = jnp.bfloat16
    pxb = px.astype(bf).astype(F32)
    pyb = py.astype(bf).astype(F32)
    pzb = pz.astype(bf).astype(F32)
    qxb = qx.astype(bf).astype(F32)
    qyb = qy.astype(bf).astype(F32)
    qzb = qz.astype(bf).astype(F32)
    csel = qxb * pxb + qyb * pyb + qzb * pzb
    dsel = q2 - 2.0 * csel + p2
    c = qx * px + qy * py + qz * pz
    d2 = q2 - 2.0 * c + p2

    inf = jnp.float32(jnp.inf)
    t = jnp.min(dsel, axis=1, keepdims=True)
    for _ in range(GROUP_SIZE - 1):
        t = jnp.min(jnp.where(dsel > t, dsel, inf), axis=1, keepdims=True)

    mask = dsel <= t
    dmin = jnp.min(jnp.where(mask, d2, inf), axis=1, keepdims=True)
    inv_sigma = inv_sigma_ref[0]
    w = jnp.where(mask, jnp.exp((dmin - d2) * inv_sigma), 0.0)
    den = jnp.sum(w, axis=1, keepdims=True)
    nx = jnp.sum(w * px, axis=1, keepdims=True)
    ny = jnp.sum(w * py, axis=1, keepdims=True)
    nz = jnp.sum(w * pz, axis=1, keepdims=True)
    r = 1.0 / den
    out_ref[0] = jnp.concatenate([nx * r, ny * r, nz * r], axis=1)


def _tc_build(B, N, MT):
    return pl.pallas_call(
        _tc_body,
        grid=(B, MT // MBLK),
        in_specs=[
            pl.BlockSpec((1, 3, N), lambda b, j: (b, 0, 0)),
            pl.BlockSpec((1, MBLK, 3), lambda b, j: (b, j, 0)),
            pl.BlockSpec(memory_space=pltpu.SMEM),
        ],
        out_specs=pl.BlockSpec((1, MBLK, 3), lambda b, j: (b, j, 0)),
        out_shape=jax.ShapeDtypeStruct((B, MT, 3), jnp.float32),
    )


def kernel(point_cloud, query_cloud, temperature):
    B, _, N = point_cloud.shape
    M = query_cloud.shape[2]
    QSC = M // 2
    sigma = jnp.maximum(temperature ** 2, jnp.asarray(MIN_SIGMA, F32))
    isig = jnp.full((L,), 1.0, F32) / sigma
    sc_out = _sc_build(B, N, M, QSC)(
        point_cloud.reshape(-1), query_cloud.reshape(-1), isig)
    qt_tc = jnp.transpose(query_cloud, (0, 2, 1))[:, QSC:, :]
    inv_sigma = (1.0 / sigma).reshape(1).astype(F32)
    tc_out = _tc_build(B, N, M - QSC)(point_cloud, qt_tc, inv_sigma)
    return jnp.concatenate(
        [sc_out.reshape(B, 3, QSC), jnp.transpose(tc_out, (0, 2, 1))], axis=2)
